# Initial kernel scaffold; baseline (speedup 1.0000x reference)
import functools

import jax
import jax.numpy as jnp
from jax import lax
from jax.experimental import pallas as pl
from jax.experimental.pallas import tpu as pltpu
from jax.experimental.pallas import tpu_sc as plsc

N_NODES = 10000
NFEAT = 128
CHUNK = 128
NC = 2
NS = 16
N_PAD = 10016


def _seg_sum_kernel(n_chunks_per_tile):
    mesh = plsc.VectorSubcoreMesh(core_axis_name="c", subcore_axis_name="s")
    edges_per_tile = n_chunks_per_tile * CHUNK
    zrows = N_PAD // NS
    orows = N_NODES // NS

    @functools.partial(
        pl.kernel,
        out_type=jax.ShapeDtypeStruct((NC, N_NODES, NFEAT), jnp.float32),
        mesh=mesh,
        scratch_types=[
            pltpu.VMEM((CHUNK,), jnp.int32),
            pltpu.VMEM((CHUNK,), jnp.int32),
            pltpu.VMEM((CHUNK, NFEAT), jnp.float32),
            pltpu.VMEM_SHARED((N_PAD, NFEAT), jnp.float32),
            pltpu.SemaphoreType.DMA,
        ],
    )
    def seg_sum(x_hbm, src_hbm, dst_hbm, zeros_hbm, out_hbm,
                src_v, dst_v, rows_v, acc_sh, sem):
        cid = lax.axis_index("c")
        sid = lax.axis_index("s")

        pltpu.sync_copy(zeros_hbm.at[pl.ds(sid * zrows, zrows)],
                        acc_sh.at[pl.ds(sid * zrows, zrows)])
        plsc.subcore_barrier()

        tile_e0 = (cid * NS + sid) * edges_per_tile

        def body(i, carry):
            base = tile_e0 + i * CHUNK
            pltpu.sync_copy(src_hbm.at[pl.ds(base, CHUNK)], src_v)
            pltpu.sync_copy(dst_hbm.at[pl.ds(base, CHUNK)], dst_v)
            pltpu.async_copy(x_hbm.at[src_v], rows_v, sem).wait()
            pltpu.sync_copy(rows_v, acc_sh.at[dst_v], add=True)
            return carry

        lax.fori_loop(0, n_chunks_per_tile, body, 0)
        plsc.subcore_barrier()

        pltpu.sync_copy(acc_sh.at[pl.ds(sid * orows, orows)],
                        out_hbm.at[cid, pl.ds(sid * orows, orows)])

    return seg_sum


def _layer1_body(pa_ref, pb_ref, w_ref, b_ref, o_ref):
    p = pa_ref[...] + pb_ref[...]
    acc = jnp.dot(p, w_ref[...], preferred_element_type=jnp.float32)
    o_ref[...] = jnp.maximum(acc + b_ref[...], 0.0)


def _final_body(pa_ref, pb_ref, x_ref, w2_ref, b2_ref, w3_ref, b3_ref,
                dsw_ref, dsb_ref, l2w_ref, l2b_ref, l3w_ref, l3b_ref,
                r1_ref, r2_ref, r3_ref):
    p = pa_ref[...] + pb_ref[...]
    c = jnp.dot(p, w2_ref[...], preferred_element_type=jnp.float32) + b2_ref[...]
    c = c - jnp.max(c, axis=1, keepdims=True)
    r1_ref[...] = c - jnp.log(jnp.sum(jnp.exp(c), axis=1, keepdims=True))
    h2 = jnp.maximum(
        jnp.dot(p, w3_ref[...], preferred_element_type=jnp.float32) + b3_ref[...],
        0.0,
    )
    h2 = h2 + jnp.dot(x_ref[...], dsw_ref[...],
                      preferred_element_type=jnp.float32) + dsb_ref[...]
    r2_ref[...] = jnp.dot(h2, l2w_ref[...],
                          preferred_element_type=jnp.float32) + l2b_ref[...]
    r3_ref[...] = jnp.dot(h2, l3w_ref[...],
                          preferred_element_type=jnp.float32) + l3b_ref[...]


_ROW_BLK = 1000


def _row_spec(cols):
    return pl.BlockSpec((_ROW_BLK, cols), lambda i: (i, 0))


def _full_spec(rows, cols):
    return pl.BlockSpec((rows, cols), lambda i: (0, 0))


def kernel(x, adj, gc1_W, gc1_b, gc2_W, gc2_b, gc3_W, gc3_b, ds_W, ds_b,
           lin2_W, lin2_b, lin3_W, lin3_b):
    n, d = x.shape
    e = adj.shape[1]
    nclass = gc2_W.shape[1]
    ndeg = lin3_W.shape[1]

    epw = NC * NS * CHUNK
    e_pad = ((e + epw - 1) // epw) * epw
    pad = e_pad - e
    src = jnp.concatenate([adj[0], jnp.zeros((pad,), jnp.int32)])
    dst = jnp.concatenate([adj[1], jnp.full((pad,), n, jnp.int32)])
    zeros_hbm = jnp.zeros((N_PAD, d), jnp.float32)

    seg_sum = _seg_sum_kernel(e_pad // (NC * NS * CHUNK))

    p1 = seg_sum(x, src, dst, zeros_hbm)

    h = pl.pallas_call(
        _layer1_body,
        grid=(n // _ROW_BLK,),
        in_specs=[_row_spec(d), _row_spec(d), _full_spec(d, d), _full_spec(1, d)],
        out_specs=_row_spec(d),
        out_shape=jax.ShapeDtypeStruct((n, d), jnp.float32),
    )(p1[0], p1[1], gc1_W, gc1_b.reshape(1, d))

    p2 = seg_sum(h, src, dst, zeros_hbm)

    r1, r2, r3 = pl.pallas_call(
        _final_body,
        grid=(n // _ROW_BLK,),
        in_specs=[
            _row_spec(d), _row_spec(d), _row_spec(d),
            _full_spec(d, nclass), _full_spec(1, nclass),
            _full_spec(d, d), _full_spec(1, d),
            _full_spec(d, d), _full_spec(1, d),
            _full_spec(d, 1), _full_spec(1, 1),
            _full_spec(d, ndeg), _full_spec(1, ndeg),
        ],
        out_specs=[_row_spec(nclass), _row_spec(1), _row_spec(ndeg)],
        out_shape=[
            jax.ShapeDtypeStruct((n, nclass), jnp.float32),
            jax.ShapeDtypeStruct((n, 1), jnp.float32),
            jax.ShapeDtypeStruct((n, ndeg), jnp.float32),
        ],
    )(p2[0], p2[1], x,
      gc2_W, gc2_b.reshape(1, nclass),
      gc3_W, gc3_b.reshape(1, d),
      ds_W, ds_b.reshape(1, d),
      lin2_W, lin2_b.reshape(1, 1),
      lin3_W, lin3_b.reshape(1, ndeg))

    return (r1, r2.squeeze(-1), r3)


# speedup vs baseline: 4.6168x; 4.6168x over previous
import functools

import jax
import jax.numpy as jnp
from jax import lax
from jax.experimental import pallas as pl
from jax.experimental.pallas import tpu as pltpu
from jax.experimental.pallas import tpu_sc as plsc

N_NODES = 10000
NFEAT = 128
CHUNK = 128
NC = 2
NS = 16
N_PAD = 10112


def _seg_sum_kernel(n_chunks_per_tile):
    mesh = plsc.VectorSubcoreMesh(core_axis_name="c", subcore_axis_name="s")
    edges_per_tile = n_chunks_per_tile * CHUNK
    rows_per_tile = N_PAD // NS

    @functools.partial(
        pl.kernel,
        out_type=jax.ShapeDtypeStruct((NC, N_PAD, NFEAT), jnp.float32),
        mesh=mesh,
        scratch_types=[
            pltpu.VMEM((CHUNK,), jnp.int32),
            pltpu.VMEM((CHUNK,), jnp.int32),
            pltpu.VMEM((CHUNK, NFEAT), jnp.float32),
            pltpu.VMEM_SHARED((N_PAD, NFEAT), jnp.float32),
            pltpu.SemaphoreType.DMA,
        ],
    )
    def seg_sum(x_hbm, src_hbm, dst_hbm, zeros_hbm, out_hbm,
                src_v, dst_v, rows_v, acc_sh, sem):
        cid = lax.axis_index("c")
        sid = lax.axis_index("s")

        pltpu.sync_copy(zeros_hbm.at[pl.ds(sid * rows_per_tile, rows_per_tile)],
                        acc_sh.at[pl.ds(sid * rows_per_tile, rows_per_tile)])
        plsc.subcore_barrier()

        tile_e0 = (cid * NS + sid) * edges_per_tile

        def body(i, carry):
            base = tile_e0 + i * CHUNK
            pltpu.sync_copy(src_hbm.at[pl.ds(base, CHUNK)], src_v)
            pltpu.sync_copy(dst_hbm.at[pl.ds(base, CHUNK)], dst_v)
            pltpu.async_copy(x_hbm.at[src_v], rows_v, sem).wait()
            pltpu.sync_copy(rows_v, acc_sh.at[dst_v], add=True)
            return carry

        lax.fori_loop(0, n_chunks_per_tile, body, 0)
        plsc.subcore_barrier()

        pltpu.sync_copy(acc_sh.at[pl.ds(sid * rows_per_tile, rows_per_tile)],
                        out_hbm.at[cid, pl.ds(sid * rows_per_tile, rows_per_tile)])

    return seg_sum


def _layer1_body(pa_ref, pb_ref, w_ref, b_ref, o_ref):
    p = pa_ref[...] + pb_ref[...]
    acc = jnp.dot(p, w_ref[...], preferred_element_type=jnp.float32)
    o_ref[...] = jnp.maximum(acc + b_ref[...], 0.0)


def _final_body(pa_ref, pb_ref, x_ref, w2_ref, b2_ref, w3_ref, b3_ref,
                dsw_ref, dsb_ref, l2w_ref, l2b_ref, l3w_ref, l3b_ref,
                r1_ref, r2_ref, r3_ref):
    p = pa_ref[...] + pb_ref[...]
    c = jnp.dot(p, w2_ref[...], preferred_element_type=jnp.float32) + b2_ref[...]
    c = c - jnp.max(c, axis=1, keepdims=True)
    r1_ref[...] = c - jnp.log(jnp.sum(jnp.exp(c), axis=1, keepdims=True))
    h2 = jnp.maximum(
        jnp.dot(p, w3_ref[...], preferred_element_type=jnp.float32) + b3_ref[...],
        0.0,
    )
    h2 = h2 + jnp.dot(x_ref[...], dsw_ref[...],
                      preferred_element_type=jnp.float32) + dsb_ref[...]
    r2_ref[...] = jnp.dot(h2, l2w_ref[...],
                          preferred_element_type=jnp.float32) + l2b_ref[...]
    r3_ref[...] = jnp.dot(h2, l3w_ref[...],
                          preferred_element_type=jnp.float32) + l3b_ref[...]


_ROW_BLK = N_PAD // 16


def _row_spec(cols):
    return pl.BlockSpec((_ROW_BLK, cols), lambda i: (i, 0))


def _full_spec(rows, cols):
    return pl.BlockSpec((rows, cols), lambda i: (0, 0))


def kernel(x, adj, gc1_W, gc1_b, gc2_W, gc2_b, gc3_W, gc3_b, ds_W, ds_b,
           lin2_W, lin2_b, lin3_W, lin3_b):
    n, d = x.shape
    e = adj.shape[1]
    nclass = gc2_W.shape[1]
    ndeg = lin3_W.shape[1]

    epw = NC * NS * CHUNK
    e_pad = ((e + epw - 1) // epw) * epw
    pad = e_pad - e
    src = jnp.concatenate([adj[0], jnp.zeros((pad,), jnp.int32)])
    dst = jnp.concatenate([adj[1], jnp.full((pad,), n, jnp.int32)])
    zeros_hbm = jnp.zeros((N_PAD, d), jnp.float32)
    x_p = jnp.concatenate([x, jnp.zeros((N_PAD - n, d), jnp.float32)])

    seg_sum = _seg_sum_kernel(e_pad // (NC * NS * CHUNK))

    p1 = seg_sum(x_p, src, dst, zeros_hbm)

    h = pl.pallas_call(
        _layer1_body,
        grid=(N_PAD // _ROW_BLK,),
        in_specs=[_row_spec(d), _row_spec(d), _full_spec(d, d), _full_spec(1, d)],
        out_specs=_row_spec(d),
        out_shape=jax.ShapeDtypeStruct((N_PAD, d), jnp.float32),
    )(p1[0], p1[1], gc1_W, gc1_b.reshape(1, d))

    p2 = seg_sum(h, src, dst, zeros_hbm)

    r1, r2, r3 = pl.pallas_call(
        _final_body,
        grid=(N_PAD // _ROW_BLK,),
        in_specs=[
            _row_spec(d), _row_spec(d), _row_spec(d),
            _full_spec(d, nclass), _full_spec(1, nclass),
            _full_spec(d, d), _full_spec(1, d),
            _full_spec(d, d), _full_spec(1, d),
            _full_spec(d, 1), _full_spec(1, 1),
            _full_spec(d, ndeg), _full_spec(1, ndeg),
        ],
        out_specs=[_row_spec(nclass), _row_spec(1), _row_spec(ndeg)],
        out_shape=[
            jax.ShapeDtypeStruct((N_PAD, nclass), jnp.float32),
            jax.ShapeDtypeStruct((N_PAD, 1), jnp.float32),
            jax.ShapeDtypeStruct((N_PAD, ndeg), jnp.float32),
        ],
    )(p2[0], p2[1], x_p,
      gc2_W, gc2_b.reshape(1, nclass),
      gc3_W, gc3_b.reshape(1, d),
      ds_W, ds_b.reshape(1, d),
      lin2_W, lin2_b.reshape(1, 1),
      lin3_W, lin3_b.reshape(1, ndeg))

    return (r1[:n], r2[:n, 0], r3[:n])
